# Initial kernel scaffold; baseline (speedup 1.0000x reference)
import jax
import jax.numpy as jnp
from jax.experimental import pallas as pl


def kernel(x, edge_index, W0, W1):
    raise NotImplementedError("write your pallas kernel here")


# speedup vs baseline: 18.1712x; 18.1712x over previous
import functools

import jax
import jax.numpy as jnp
from jax import lax
from jax.experimental import pallas as pl
from jax.experimental.pallas import tpu as pltpu
from jax.experimental.pallas import tpu_sc as plsc

N = 10000
D_IN = 256
D_HID = 512
NL = 40
NLP = 64
E = 160000

NC = 2
NS = 16
NP = 10240
RS = NP // NS
EP = 163840
PAD = EP - E
CHUNK = 128
ER = EP // CHUNK
CH1 = EP // NS // CHUNK
CH2 = EP // (NC * NS) // CHUNK

RB = 512
NPB = NP // RB

_mesh = plsc.VectorSubcoreMesh(core_axis_name="c", subcore_axis_name="s")


@functools.partial(
    pl.kernel,
    out_type=jax.ShapeDtypeStruct((2 * NP, 16), jnp.float32),
    mesh=_mesh,
    compiler_params=pltpu.CompilerParams(use_tc_tiling_on_sc=False),
    scratch_types=[
        pltpu.VMEM((CH2, CHUNK), jnp.int32),
        pltpu.VMEM((CHUNK, 16), jnp.float32),
        pltpu.VMEM_SHARED((NP, 16), jnp.float32),
    ],
)
def _deg_kernel(dst_hbm, ones_hbm, zeros_hbm, out_hbm, idx_v, ones_v, acc_sh):
    c = lax.axis_index("c")
    s = lax.axis_index("s")
    w = c * NS + s
    pltpu.sync_copy(zeros_hbm, acc_sh.at[pl.ds(s * RS, RS)])
    pltpu.sync_copy(dst_hbm.at[pl.ds(w * CH2, CH2)], idx_v)
    pltpu.sync_copy(ones_hbm, ones_v)
    plsc.subcore_barrier()

    def body(j, carry):
        pltpu.sync_copy(ones_v, acc_sh.at[idx_v.at[j]], add=True)
        return carry

    lax.fori_loop(0, CH2, body, 0)
    plsc.subcore_barrier()
    pltpu.sync_copy(acc_sh.at[pl.ds(s * RS, RS)],
                    out_hbm.at[pl.ds(c * NP + s * RS, RS)])


def _make_prop_kernel(chunks_per_sub):
    @functools.partial(
        pl.kernel,
        out_type=jax.ShapeDtypeStruct((2 * NP, NLP), jnp.float32),
        mesh=_mesh,
        compiler_params=pltpu.CompilerParams(use_tc_tiling_on_sc=False),
        scratch_types=[
            pltpu.VMEM((chunks_per_sub, CHUNK), jnp.int32),
            pltpu.VMEM((chunks_per_sub, CHUNK), jnp.int32),
            pltpu.VMEM((2, CHUNK, NLP), jnp.float32),
            pltpu.VMEM_SHARED((NP, NLP), jnp.float32),
            pltpu.SemaphoreType.DMA,
            pltpu.SemaphoreType.DMA,
        ],
    )
    def prop(rows_hbm, src_hbm, dst_hbm, zeros_hbm, out_hbm,
             si_v, di_v, buf_v, acc_sh, sem0, sem1):
        c = lax.axis_index("c")
        s = lax.axis_index("s")
        src_row0 = (c * NS + s) * chunks_per_sub
        pltpu.sync_copy(zeros_hbm, acc_sh.at[pl.ds(s * RS, RS)])
        pltpu.sync_copy(src_hbm.at[pl.ds(src_row0, chunks_per_sub)], si_v)
        pltpu.sync_copy(dst_hbm.at[pl.ds(src_row0, chunks_per_sub)], di_v)
        plsc.subcore_barrier()

        pltpu.async_copy(rows_hbm.at[si_v.at[0]], buf_v.at[0], sem0)

        def body(i, carry):
            j0 = 2 * i
            pltpu.async_copy(rows_hbm.at[si_v.at[j0 + 1]], buf_v.at[1], sem1)
            pltpu.make_async_copy(rows_hbm.at[si_v.at[j0]],
                                  buf_v.at[0], sem0).wait()
            pltpu.sync_copy(buf_v.at[0], acc_sh.at[di_v.at[j0]], add=True)

            @pl.when(i < chunks_per_sub // 2 - 1)
            def _():
                pltpu.async_copy(rows_hbm.at[si_v.at[j0 + 2]],
                                 buf_v.at[0], sem0)

            pltpu.make_async_copy(rows_hbm.at[si_v.at[j0 + 1]],
                                  buf_v.at[1], sem1).wait()
            pltpu.sync_copy(buf_v.at[1], acc_sh.at[di_v.at[j0 + 1]], add=True)
            return carry

        lax.fori_loop(0, chunks_per_sub // 2, body, 0)
        plsc.subcore_barrier()
        pltpu.sync_copy(acc_sh.at[pl.ds(s * RS, RS)],
                        out_hbm.at[pl.ds(c * NP + s * RS, RS)])

    return prop


_l1_kernel = _make_prop_kernel(CH1)
_l2_kernel = _make_prop_kernel(CH2)


def _dinv_of(d0, d1):
    return lax.rsqrt(d0[:, 0:1] + d1[:, 0:1] + 1.0)


def _prep_body(x_ref, d0_ref, d1_ref, oa_ref, ob_ref):
    dinv = _dinv_of(d0_ref, d1_ref)
    oa_ref[...] = x_ref[:, 0:NLP] * dinv
    ob_ref[...] = x_ref[:, NLP:2 * NLP] * dinv


def _mm_body(a0, b0, a1, b1, x0, x1, x2, x3, d0, d1, w0, w1, o_ref):
    dinv = _dinv_of(d0, d1)
    p0 = (a0[...] + x0[...]) * dinv
    p1 = (b0[...] + x1[...]) * dinv
    p2 = (a1[...] + x2[...]) * dinv
    p3 = (b1[...] + x3[...]) * dinv
    f = jnp.float32
    h = (jnp.dot(p0, w0[0:64, :], preferred_element_type=f)
         + jnp.dot(p1, w0[64:128, :], preferred_element_type=f)
         + jnp.dot(p2, w0[128:192, :], preferred_element_type=f)
         + jnp.dot(p3, w0[192:256, :], preferred_element_type=f))
    h = jnp.maximum(h, 0.0)
    g = jnp.dot(h, w1[...], preferred_element_type=f)
    o_ref[...] = g * dinv


def _fin_body(a0, a1, g, d0, d1, o_ref):
    t = (a0[...] + a1[...] + g[...]) * _dinv_of(d0, d1)
    col = lax.broadcasted_iota(jnp.int32, (RB, NLP), 1)
    t = jnp.where(col < NL, t, -jnp.inf)
    m = jnp.max(t, axis=1, keepdims=True)
    e = jnp.exp(t - m)
    o_ref[...] = e / jnp.sum(e, axis=1, keepdims=True)


def kernel(x, edge_index, W0, W1):
    f32 = jnp.float32
    src = edge_index[0].astype(jnp.int32)
    dst = edge_index[1].astype(jnp.int32)

    spread = jnp.arange(PAD, dtype=jnp.int32) % (NP - N)
    pad_zero0 = N + spread
    pad_zero1 = NP + N + spread
    pad_trash = N + spread
    dst_pad = jnp.concatenate([dst, pad_trash])
    dst2d = dst_pad.reshape(ER, CHUNK)
    dst2d_l1 = jnp.concatenate([dst_pad, dst_pad]).reshape(2 * ER, CHUNK)
    src2d_l1 = jnp.concatenate(
        [src, pad_zero0, src + NP, pad_zero1]).reshape(2 * ER, CHUNK)
    src2d_l2 = jnp.concatenate([src, pad_zero0]).reshape(ER, CHUNK)

    x_pad = jnp.pad(x, ((0, NP - N), (0, 0)))
    W1p = jnp.pad(W1, ((0, 0), (0, NLP - NL)))

    ones_deg = jnp.ones((CHUNK, 16), f32)
    zeros_deg = jnp.zeros((RS, 16), f32)
    zeros_64 = jnp.zeros((RS, NLP), f32)

    degs = _deg_kernel(dst2d, ones_deg, zeros_deg)
    d0, d1 = degs[:NP], degs[NP:]

    xsA, xsB = pl.pallas_call(
        _prep_body,
        grid=(2, NPB),
        in_specs=[
            pl.BlockSpec((RB, 2 * NLP), lambda j, i: (i, j)),
            pl.BlockSpec((RB, 16), lambda j, i: (i, 0)),
            pl.BlockSpec((RB, 16), lambda j, i: (i, 0)),
        ],
        out_specs=[
            pl.BlockSpec((RB, NLP), lambda j, i: (j * NPB + i, 0)),
            pl.BlockSpec((RB, NLP), lambda j, i: (j * NPB + i, 0)),
        ],
        out_shape=[
            jax.ShapeDtypeStruct((2 * NP, NLP), f32),
            jax.ShapeDtypeStruct((2 * NP, NLP), f32),
        ],
    )(x_pad, d0, d1)

    acc1A = _l1_kernel(xsA, src2d_l1, dst2d_l1, zeros_64)
    acc1B = _l1_kernel(xsB, src2d_l1, dst2d_l1, zeros_64)

    gs = pl.pallas_call(
        _mm_body,
        grid=(NPB,),
        in_specs=[
            pl.BlockSpec((RB, NLP), lambda i: (i, 0)),
            pl.BlockSpec((RB, NLP), lambda i: (i, 0)),
            pl.BlockSpec((RB, NLP), lambda i: (NPB + i, 0)),
            pl.BlockSpec((RB, NLP), lambda i: (NPB + i, 0)),
            pl.BlockSpec((RB, NLP), lambda i: (i, 0)),
            pl.BlockSpec((RB, NLP), lambda i: (i, 0)),
            pl.BlockSpec((RB, NLP), lambda i: (NPB + i, 0)),
            pl.BlockSpec((RB, NLP), lambda i: (NPB + i, 0)),
            pl.BlockSpec((RB, 16), lambda i: (i, 0)),
            pl.BlockSpec((RB, 16), lambda i: (i, 0)),
            pl.BlockSpec((D_IN, D_HID), lambda i: (0, 0)),
            pl.BlockSpec((D_HID, NLP), lambda i: (0, 0)),
        ],
        out_specs=pl.BlockSpec((RB, NLP), lambda i: (i, 0)),
        out_shape=jax.ShapeDtypeStruct((NP, NLP), f32),
    )(acc1A, acc1B, acc1A, acc1B, xsA, xsB, xsA, xsB, d0, d1, W0, W1p)

    acc2 = _l2_kernel(gs, src2d_l2, dst2d, zeros_64)

    res = pl.pallas_call(
        _fin_body,
        grid=(NPB,),
        in_specs=[
            pl.BlockSpec((RB, NLP), lambda i: (i, 0)),
            pl.BlockSpec((RB, NLP), lambda i: (NPB + i, 0)),
            pl.BlockSpec((RB, NLP), lambda i: (i, 0)),
            pl.BlockSpec((RB, 16), lambda i: (i, 0)),
            pl.BlockSpec((RB, 16), lambda i: (i, 0)),
        ],
        out_specs=pl.BlockSpec((RB, NLP), lambda i: (i, 0)),
        out_shape=jax.ShapeDtypeStruct((NP, NLP), f32),
    )(acc2, acc2, gs, d0, d1)

    return res[:N, :NL]


# speedup vs baseline: 19.5224x; 1.0744x over previous
import functools

import jax
import jax.numpy as jnp
from jax import lax
from jax.experimental import pallas as pl
from jax.experimental.pallas import tpu as pltpu
from jax.experimental.pallas import tpu_sc as plsc

N = 10000
D_IN = 256
D_HID = 512
NL = 40
W1W = 64
W2 = 48
E = 160000

NC = 2
NS = 16
NP = 10240
RS = NP // NS
EP = 163840
PAD = EP - E
CHUNK = 128
ER = EP // CHUNK
CH1 = EP // NS // CHUNK
CH2 = EP // (NC * NS) // CHUNK
NBUF = 4

RB = 512
NPB = NP // RB

_mesh = plsc.VectorSubcoreMesh(core_axis_name="c", subcore_axis_name="s")


@functools.partial(
    pl.kernel,
    out_type=jax.ShapeDtypeStruct((2 * NP, 16), jnp.float32),
    mesh=_mesh,
    compiler_params=pltpu.CompilerParams(use_tc_tiling_on_sc=False),
    scratch_types=[
        pltpu.VMEM((CH2, CHUNK), jnp.int32),
        pltpu.VMEM((CHUNK, 16), jnp.float32),
        pltpu.VMEM_SHARED((NP, 16), jnp.float32),
    ],
)
def _deg_kernel(dst_hbm, ones_hbm, zeros_hbm, out_hbm, idx_v, ones_v, acc_sh):
    c = lax.axis_index("c")
    s = lax.axis_index("s")
    w = c * NS + s
    pltpu.sync_copy(zeros_hbm, acc_sh.at[pl.ds(s * RS, RS)])
    pltpu.sync_copy(dst_hbm.at[pl.ds(w * CH2, CH2)], idx_v)
    pltpu.sync_copy(ones_hbm, ones_v)
    plsc.subcore_barrier()

    def body(j, carry):
        pltpu.sync_copy(ones_v, acc_sh.at[idx_v.at[j]], add=True)
        return carry

    lax.fori_loop(0, CH2, body, 0)
    plsc.subcore_barrier()
    pltpu.sync_copy(acc_sh.at[pl.ds(s * RS, RS)],
                    out_hbm.at[pl.ds(c * NP + s * RS, RS)])


def _edge_loop(rows_hbm, si_v, di_v, acc_sh, buf_v, gsems, ssems, cps):
    bufs = [buf_v.at[b] for b in range(NBUF)]

    for b in range(NBUF):
        pltpu.async_copy(rows_hbm.at[si_v.at[b]], bufs[b], gsems[b])

    def body(i, carry):
        for b in range(NBUF):
            j = i * NBUF + b
            pltpu.make_async_copy(rows_hbm.at[si_v.at[j]],
                                  bufs[b], gsems[b]).wait()
            pltpu.async_copy(bufs[b], acc_sh.at[di_v.at[j]], ssems[b],
                             add=True)
        for b in range(NBUF):
            j = i * NBUF + b

            @pl.when(j + NBUF < cps)
            def _():
                pltpu.make_async_copy(bufs[b], acc_sh.at[di_v.at[j]],
                                      ssems[b]).wait()
                pltpu.async_copy(rows_hbm.at[si_v.at[j + NBUF]],
                                 bufs[b], gsems[b])
        return carry

    lax.fori_loop(0, cps // NBUF, body, 0)
    for b in range(NBUF):
        j = cps - NBUF + b
        pltpu.make_async_copy(bufs[b], acc_sh.at[di_v.at[j]],
                              ssems[b]).wait()


def _make_prop_kernel(width, cps, nphases, tc_tiling):
    sems = [pltpu.SemaphoreType.DMA] * (2 * NBUF)

    @functools.partial(
        pl.kernel,
        out_type=[jax.ShapeDtypeStruct((2 * NP, width), jnp.float32)
                  for _ in range(nphases)],
        mesh=_mesh,
        compiler_params=pltpu.CompilerParams(use_tc_tiling_on_sc=tc_tiling),
        scratch_types=[
            pltpu.VMEM((cps, CHUNK), jnp.int32),
            pltpu.VMEM((cps, CHUNK), jnp.int32),
            pltpu.VMEM((NBUF, CHUNK, width), jnp.float32),
            pltpu.VMEM_SHARED((NP, width), jnp.float32),
        ] + sems,
    )
    def prop(*args):
        rows_list = args[:nphases]
        src_hbm, dst_hbm, zeros_hbm = args[nphases:nphases + 3]
        outs = args[nphases + 3:2 * nphases + 3]
        si_v, di_v, buf_v, acc_sh = args[2 * nphases + 3:2 * nphases + 7]
        sems = args[2 * nphases + 7:]
        gsems, ssems = sems[:NBUF], sems[NBUF:]
        c = lax.axis_index("c")
        s = lax.axis_index("s")
        row0 = (c * NS + s) * cps
        pltpu.sync_copy(zeros_hbm, acc_sh.at[pl.ds(s * RS, RS)])
        pltpu.sync_copy(src_hbm.at[pl.ds(row0, cps)], si_v)
        pltpu.sync_copy(dst_hbm.at[pl.ds(row0, cps)], di_v)
        plsc.subcore_barrier()

        for p in range(nphases):
            _edge_loop(rows_list[p], si_v, di_v, acc_sh, buf_v,
                       gsems, ssems, cps)
            plsc.subcore_barrier()
            pltpu.sync_copy(acc_sh.at[pl.ds(s * RS, RS)],
                            outs[p].at[pl.ds(c * NP + s * RS, RS)])
            if p + 1 < nphases:
                pltpu.sync_copy(zeros_hbm, acc_sh.at[pl.ds(s * RS, RS)])
                plsc.subcore_barrier()

    return prop


_l1_kernel = _make_prop_kernel(W1W, CH1, nphases=2, tc_tiling=False)
_l2_kernel = _make_prop_kernel(W2, CH2, nphases=1, tc_tiling=False)


def _dinv_of(d0, d1):
    return lax.rsqrt(d0[:, 0:1] + d1[:, 0:1] + 1.0)


def _prep_body(x_ref, d0_ref, d1_ref, oa_ref, ob_ref):
    dinv = _dinv_of(d0_ref, d1_ref)
    oa_ref[...] = x_ref[:, 0:W1W] * dinv
    ob_ref[...] = x_ref[:, W1W:2 * W1W] * dinv


def _mm_body(a0, b0, a1, b1, x0, x1, x2, x3, d0, d1, w0, w1, o_ref):
    dinv = _dinv_of(d0, d1)
    p0 = (a0[...] + x0[...]) * dinv
    p1 = (b0[...] + x1[...]) * dinv
    p2 = (a1[...] + x2[...]) * dinv
    p3 = (b1[...] + x3[...]) * dinv
    f = jnp.float32
    h = (jnp.dot(p0, w0[0:64, :], preferred_element_type=f)
         + jnp.dot(p1, w0[64:128, :], preferred_element_type=f)
         + jnp.dot(p2, w0[128:192, :], preferred_element_type=f)
         + jnp.dot(p3, w0[192:256, :], preferred_element_type=f))
    h = jnp.maximum(h, 0.0)
    g = jnp.dot(h, w1[...], preferred_element_type=f)
    o_ref[...] = g * dinv


def _fin_body(a0, a1, g, d0, d1, o_ref):
    t = (a0[...] + a1[...] + g[...]) * _dinv_of(d0, d1)
    col = lax.broadcasted_iota(jnp.int32, (RB, W2), 1)
    t = jnp.where(col < NL, t, -jnp.inf)
    m = jnp.max(t, axis=1, keepdims=True)
    e = jnp.exp(t - m)
    o_ref[...] = e / jnp.sum(e, axis=1, keepdims=True)


def kernel(x, edge_index, W0, W1):
    f32 = jnp.float32
    src = edge_index[0].astype(jnp.int32)
    dst = edge_index[1].astype(jnp.int32)

    spread = jnp.arange(PAD, dtype=jnp.int32) % (NP - N)
    pad_row0 = N + spread
    pad_row1 = NP + N + spread
    dst_pad = jnp.concatenate([dst, pad_row0])
    dst2d = dst_pad.reshape(ER, CHUNK)
    dst2d_l1 = jnp.concatenate([dst_pad, dst_pad]).reshape(2 * ER, CHUNK)
    src2d_l1 = jnp.concatenate(
        [src, pad_row0, src + NP, pad_row1]).reshape(2 * ER, CHUNK)
    src2d_l2 = jnp.concatenate([src, pad_row0]).reshape(ER, CHUNK)

    W1p = jnp.pad(W1, ((0, 0), (0, W2 - NL)))

    ones_deg = jnp.ones((CHUNK, 16), f32)
    zeros_deg = jnp.zeros((RS, 16), f32)
    zeros_w1 = jnp.zeros((RS, W1W), f32)
    zeros_w2 = jnp.zeros((RS, W2), f32)

    degs = _deg_kernel(dst2d, ones_deg, zeros_deg)
    d0, d1 = degs[:NP], degs[NP:]

    xsA, xsB = pl.pallas_call(
        _prep_body,
        grid=(2, NPB),
        in_specs=[
            pl.BlockSpec((RB, 2 * W1W), lambda j, i: (i, j)),
            pl.BlockSpec((RB, 16), lambda j, i: (i, 0)),
            pl.BlockSpec((RB, 16), lambda j, i: (i, 0)),
        ],
        out_specs=[
            pl.BlockSpec((RB, W1W), lambda j, i: (j * NPB + i, 0)),
            pl.BlockSpec((RB, W1W), lambda j, i: (j * NPB + i, 0)),
        ],
        out_shape=[
            jax.ShapeDtypeStruct((2 * NP, W1W), f32),
            jax.ShapeDtypeStruct((2 * NP, W1W), f32),
        ],
    )(x, d0, d1)

    acc1A, acc1B = _l1_kernel(xsA, xsB, src2d_l1, dst2d_l1, zeros_w1)

    gs = pl.pallas_call(
        _mm_body,
        grid=(NPB,),
        in_specs=[
            pl.BlockSpec((RB, W1W), lambda i: (i, 0)),
            pl.BlockSpec((RB, W1W), lambda i: (i, 0)),
            pl.BlockSpec((RB, W1W), lambda i: (NPB + i, 0)),
            pl.BlockSpec((RB, W1W), lambda i: (NPB + i, 0)),
            pl.BlockSpec((RB, W1W), lambda i: (i, 0)),
            pl.BlockSpec((RB, W1W), lambda i: (i, 0)),
            pl.BlockSpec((RB, W1W), lambda i: (NPB + i, 0)),
            pl.BlockSpec((RB, W1W), lambda i: (NPB + i, 0)),
            pl.BlockSpec((RB, 16), lambda i: (i, 0)),
            pl.BlockSpec((RB, 16), lambda i: (i, 0)),
            pl.BlockSpec((D_IN, D_HID), lambda i: (0, 0)),
            pl.BlockSpec((D_HID, W2), lambda i: (0, 0)),
        ],
        out_specs=pl.BlockSpec((RB, W2), lambda i: (i, 0)),
        out_shape=jax.ShapeDtypeStruct((NP, W2), f32),
    )(acc1A, acc1B, acc1A, acc1B, xsA, xsB, xsA, xsB, d0, d1, W0, W1p)

    (acc2,) = _l2_kernel(gs, src2d_l2, dst2d, zeros_w2)

    res = pl.pallas_call(
        _fin_body,
        grid=(NPB,),
        in_specs=[
            pl.BlockSpec((RB, W2), lambda i: (i, 0)),
            pl.BlockSpec((RB, W2), lambda i: (NPB + i, 0)),
            pl.BlockSpec((RB, W2), lambda i: (i, 0)),
            pl.BlockSpec((RB, 16), lambda i: (i, 0)),
            pl.BlockSpec((RB, 16), lambda i: (i, 0)),
        ],
        out_specs=pl.BlockSpec((RB, W2), lambda i: (i, 0)),
        out_shape=jax.ShapeDtypeStruct((NP, W2), f32),
    )(acc2, acc2, gs, d0, d1)

    return res[:N, :NL]


# speedup vs baseline: 21.4026x; 1.0963x over previous
import functools

import jax
import jax.numpy as jnp
from jax import lax
from jax.experimental import pallas as pl
from jax.experimental.pallas import tpu as pltpu
from jax.experimental.pallas import tpu_sc as plsc

N = 10000
D_IN = 256
D_HID = 512
NL = 40
W1W = 64
W2 = 48
E = 160000

NC = 2
NS = 16
NP = 10240
RS = NP // NS
EP = 163840
PAD = EP - E
CHUNK = 128
ER = EP // CHUNK
CH1 = EP // NS // CHUNK
CH2 = EP // (NC * NS) // CHUNK
NBUF = 8

RB = 1024
NPB = NP // RB

_mesh = plsc.VectorSubcoreMesh(core_axis_name="c", subcore_axis_name="s")


@functools.partial(
    pl.kernel,
    out_type=jax.ShapeDtypeStruct((2 * NP, 16), jnp.float32),
    mesh=_mesh,
    compiler_params=pltpu.CompilerParams(use_tc_tiling_on_sc=False),
    scratch_types=[
        pltpu.VMEM((CH2, CHUNK), jnp.int32),
        pltpu.VMEM((CHUNK, 16), jnp.float32),
        pltpu.VMEM_SHARED((NP, 16), jnp.float32),
    ],
)
def _deg_kernel(dst_hbm, ones_hbm, zeros_hbm, out_hbm, idx_v, ones_v, acc_sh):
    c = lax.axis_index("c")
    s = lax.axis_index("s")
    w = c * NS + s
    pltpu.sync_copy(zeros_hbm, acc_sh.at[pl.ds(s * RS, RS)])
    pltpu.sync_copy(dst_hbm.at[pl.ds(w * CH2, CH2)], idx_v)
    pltpu.sync_copy(ones_hbm, ones_v)
    plsc.subcore_barrier()

    def body(j, carry):
        pltpu.sync_copy(ones_v, acc_sh.at[idx_v.at[j]], add=True)
        return carry

    lax.fori_loop(0, CH2, body, 0)
    plsc.subcore_barrier()
    pltpu.sync_copy(acc_sh.at[pl.ds(s * RS, RS)],
                    out_hbm.at[pl.ds(c * NP + s * RS, RS)])


def _edge_loop(rows_hbm, si_v, di_v, acc_sh, buf_v, gsems, ssems, cps):
    bufs = [buf_v.at[b] for b in range(NBUF)]

    for b in range(NBUF):
        pltpu.async_copy(rows_hbm.at[si_v.at[b]], bufs[b], gsems[b])

    def body(i, carry):
        for b in range(NBUF):
            j = i * NBUF + b
            pltpu.make_async_copy(rows_hbm.at[si_v.at[j]],
                                  bufs[b], gsems[b]).wait()
            pltpu.async_copy(bufs[b], acc_sh.at[di_v.at[j]], ssems[b],
                             add=True)
        for b in range(NBUF):
            j = i * NBUF + b

            @pl.when(j + NBUF < cps)
            def _():
                pltpu.make_async_copy(bufs[b], acc_sh.at[di_v.at[j]],
                                      ssems[b]).wait()
                pltpu.async_copy(rows_hbm.at[si_v.at[j + NBUF]],
                                 bufs[b], gsems[b])
        return carry

    lax.fori_loop(0, cps // NBUF, body, 0)
    for b in range(NBUF):
        j = cps - NBUF + b
        pltpu.make_async_copy(bufs[b], acc_sh.at[di_v.at[j]],
                              ssems[b]).wait()


def _make_prop_kernel(width, cps, nphases, tc_tiling):
    sems = [pltpu.SemaphoreType.DMA] * (2 * NBUF)

    @functools.partial(
        pl.kernel,
        out_type=[jax.ShapeDtypeStruct((2 * NP, width), jnp.float32)
                  for _ in range(nphases)],
        mesh=_mesh,
        compiler_params=pltpu.CompilerParams(use_tc_tiling_on_sc=tc_tiling),
        scratch_types=[
            pltpu.VMEM((cps, CHUNK), jnp.int32),
            pltpu.VMEM((cps, CHUNK), jnp.int32),
            pltpu.VMEM((NBUF, CHUNK, width), jnp.float32),
            pltpu.VMEM_SHARED((NP, width), jnp.float32),
        ] + sems,
    )
    def prop(*args):
        rows_list = args[:nphases]
        src_hbm, dst_hbm, zeros_hbm = args[nphases:nphases + 3]
        outs = args[nphases + 3:2 * nphases + 3]
        si_v, di_v, buf_v, acc_sh = args[2 * nphases + 3:2 * nphases + 7]
        sems = args[2 * nphases + 7:]
        gsems, ssems = sems[:NBUF], sems[NBUF:]
        c = lax.axis_index("c")
        s = lax.axis_index("s")
        row0 = (c * NS + s) * cps
        pltpu.sync_copy(zeros_hbm, acc_sh.at[pl.ds(s * RS, RS)])
        pltpu.sync_copy(src_hbm.at[pl.ds(row0, cps)], si_v)
        pltpu.sync_copy(dst_hbm.at[pl.ds(row0, cps)], di_v)
        plsc.subcore_barrier()

        for p in range(nphases):
            _edge_loop(rows_list[p], si_v, di_v, acc_sh, buf_v,
                       gsems, ssems, cps)
            plsc.subcore_barrier()
            pltpu.sync_copy(acc_sh.at[pl.ds(s * RS, RS)],
                            outs[p].at[pl.ds(c * NP + s * RS, RS)])
            if p + 1 < nphases:
                pltpu.sync_copy(zeros_hbm, acc_sh.at[pl.ds(s * RS, RS)])
                plsc.subcore_barrier()

    return prop


_l1_kernel = _make_prop_kernel(W1W, CH1, nphases=2, tc_tiling=False)
_l2_kernel = _make_prop_kernel(W2, CH2, nphases=1, tc_tiling=False)


def _dinv_of(d0, d1):
    return lax.rsqrt(d0[:, 0:1] + d1[:, 0:1] + 1.0)


def _prep_body(x_ref, d0_ref, d1_ref, oa_ref, ob_ref):
    dinv = _dinv_of(d0_ref, d1_ref)
    oa_ref[...] = x_ref[:, 0:W1W] * dinv
    ob_ref[...] = x_ref[:, W1W:2 * W1W] * dinv


def _mm_body(a0, b0, a1, b1, x0, x1, x2, x3, d0, d1, w0, w1, o_ref):
    dinv = _dinv_of(d0, d1)
    p0 = (a0[...] + x0[...]) * dinv
    p1 = (b0[...] + x1[...]) * dinv
    p2 = (a1[...] + x2[...]) * dinv
    p3 = (b1[...] + x3[...]) * dinv
    f = jnp.float32
    h = (jnp.dot(p0, w0[0:64, :], preferred_element_type=f)
         + jnp.dot(p1, w0[64:128, :], preferred_element_type=f)
         + jnp.dot(p2, w0[128:192, :], preferred_element_type=f)
         + jnp.dot(p3, w0[192:256, :], preferred_element_type=f))
    h = jnp.maximum(h, 0.0)
    g = jnp.dot(h, w1[...], preferred_element_type=f)
    o_ref[...] = g * dinv


def _fin_body(a0, a1, g, d0, d1, o_ref):
    t = (a0[...] + a1[...] + g[...]) * _dinv_of(d0, d1)
    col = lax.broadcasted_iota(jnp.int32, (RB, W2), 1)
    t = jnp.where(col < NL, t, -jnp.inf)
    m = jnp.max(t, axis=1, keepdims=True)
    e = jnp.exp(t - m)
    o_ref[...] = e / jnp.sum(e, axis=1, keepdims=True)


def kernel(x, edge_index, W0, W1):
    f32 = jnp.float32
    src = edge_index[0].astype(jnp.int32)
    dst = edge_index[1].astype(jnp.int32)

    spread = jnp.arange(PAD, dtype=jnp.int32) % (NP - N)
    pad_row0 = N + spread
    pad_row1 = NP + N + spread
    dst_pad = jnp.concatenate([dst, pad_row0])
    dst2d = dst_pad.reshape(ER, CHUNK)
    dst2d_l1 = jnp.concatenate([dst_pad, dst_pad]).reshape(2 * ER, CHUNK)
    src2d_l1 = jnp.concatenate(
        [src, pad_row0, src + NP, pad_row1]).reshape(2 * ER, CHUNK)
    src2d_l2 = jnp.concatenate([src, pad_row0]).reshape(ER, CHUNK)

    W1p = jnp.pad(W1, ((0, 0), (0, W2 - NL)))

    ones_deg = jnp.ones((CHUNK, 16), f32)
    zeros_deg = jnp.zeros((RS, 16), f32)
    zeros_w1 = jnp.zeros((RS, W1W), f32)
    zeros_w2 = jnp.zeros((RS, W2), f32)

    degs = _deg_kernel(dst2d, ones_deg, zeros_deg)
    d0, d1 = degs[:NP], degs[NP:]

    xsA, xsB = pl.pallas_call(
        _prep_body,
        grid=(2, NPB),
        in_specs=[
            pl.BlockSpec((RB, 2 * W1W), lambda j, i: (i, j)),
            pl.BlockSpec((RB, 16), lambda j, i: (i, 0)),
            pl.BlockSpec((RB, 16), lambda j, i: (i, 0)),
        ],
        out_specs=[
            pl.BlockSpec((RB, W1W), lambda j, i: (j * NPB + i, 0)),
            pl.BlockSpec((RB, W1W), lambda j, i: (j * NPB + i, 0)),
        ],
        out_shape=[
            jax.ShapeDtypeStruct((2 * NP, W1W), f32),
            jax.ShapeDtypeStruct((2 * NP, W1W), f32),
        ],
    )(x, d0, d1)

    acc1A, acc1B = _l1_kernel(xsA, xsB, src2d_l1, dst2d_l1, zeros_w1)

    gs = pl.pallas_call(
        _mm_body,
        grid=(NPB,),
        in_specs=[
            pl.BlockSpec((RB, W1W), lambda i: (i, 0)),
            pl.BlockSpec((RB, W1W), lambda i: (i, 0)),
            pl.BlockSpec((RB, W1W), lambda i: (NPB + i, 0)),
            pl.BlockSpec((RB, W1W), lambda i: (NPB + i, 0)),
            pl.BlockSpec((RB, W1W), lambda i: (i, 0)),
            pl.BlockSpec((RB, W1W), lambda i: (i, 0)),
            pl.BlockSpec((RB, W1W), lambda i: (NPB + i, 0)),
            pl.BlockSpec((RB, W1W), lambda i: (NPB + i, 0)),
            pl.BlockSpec((RB, 16), lambda i: (i, 0)),
            pl.BlockSpec((RB, 16), lambda i: (i, 0)),
            pl.BlockSpec((D_IN, D_HID), lambda i: (0, 0)),
            pl.BlockSpec((D_HID, W2), lambda i: (0, 0)),
        ],
        out_specs=pl.BlockSpec((RB, W2), lambda i: (i, 0)),
        out_shape=jax.ShapeDtypeStruct((NP, W2), f32),
    )(acc1A, acc1B, acc1A, acc1B, xsA, xsB, xsA, xsB, d0, d1, W0, W1p)

    (acc2,) = _l2_kernel(gs, src2d_l2, dst2d, zeros_w2)

    res = pl.pallas_call(
        _fin_body,
        grid=(NPB,),
        in_specs=[
            pl.BlockSpec((RB, W2), lambda i: (i, 0)),
            pl.BlockSpec((RB, W2), lambda i: (NPB + i, 0)),
            pl.BlockSpec((RB, W2), lambda i: (i, 0)),
            pl.BlockSpec((RB, 16), lambda i: (i, 0)),
            pl.BlockSpec((RB, 16), lambda i: (i, 0)),
        ],
        out_specs=pl.BlockSpec((RB, W2), lambda i: (i, 0)),
        out_shape=jax.ShapeDtypeStruct((NP, W2), f32),
    )(acc2, acc2, gs, d0, d1)

    return res[:N, :NL]


# speedup vs baseline: 21.8207x; 1.0195x over previous
import functools

import jax
import jax.numpy as jnp
from jax import lax
from jax.experimental import pallas as pl
from jax.experimental.pallas import tpu as pltpu
from jax.experimental.pallas import tpu_sc as plsc

N = 10000
D_IN = 256
D_HID = 512
NL = 40
W1W = 64
W2 = 48
E = 160000

NC = 2
NS = 16
NP = 10240
RS = NP // NS
EP = 163840
PAD = EP - E
CHUNK = 128
ER = EP // CHUNK
CH1 = EP // NS // CHUNK
CH2 = EP // (NC * NS) // CHUNK
NBUF = 8

RB = 1024
NPB = NP // RB

_mesh = plsc.VectorSubcoreMesh(core_axis_name="c", subcore_axis_name="s")


@functools.partial(
    pl.kernel,
    out_type=jax.ShapeDtypeStruct((2 * NP, 16), jnp.float32),
    mesh=_mesh,
    compiler_params=pltpu.CompilerParams(use_tc_tiling_on_sc=False),
    scratch_types=[
        pltpu.VMEM((CH2, CHUNK), jnp.int32),
        pltpu.VMEM((CHUNK, 16), jnp.float32),
        pltpu.VMEM_SHARED((NP, 16), jnp.float32),
    ],
)
def _deg_kernel(dst_hbm, ones_hbm, zeros_hbm, out_hbm, idx_v, ones_v, acc_sh):
    c = lax.axis_index("c")
    s = lax.axis_index("s")
    w = c * NS + s
    pltpu.sync_copy(zeros_hbm, acc_sh.at[pl.ds(s * RS, RS)])
    pltpu.sync_copy(dst_hbm.at[pl.ds(w * CH2, CH2)], idx_v)
    pltpu.sync_copy(ones_hbm, ones_v)
    plsc.subcore_barrier()

    def body(j, carry):
        pltpu.sync_copy(ones_v, acc_sh.at[idx_v.at[j]], add=True)
        return carry

    lax.fori_loop(0, CH2, body, 0)
    plsc.subcore_barrier()
    pltpu.sync_copy(acc_sh.at[pl.ds(s * RS, RS)],
                    out_hbm.at[pl.ds(c * NP + s * RS, RS)])


def _edge_loop(rows_hbm, si_v, di_v, acc_sh, buf_v, gsems, ssems, cps):
    bufs = [buf_v.at[b] for b in range(NBUF)]

    for b in range(NBUF):
        pltpu.async_copy(rows_hbm.at[si_v.at[b]], bufs[b], gsems[b])

    def body(i, carry):
        for b in range(NBUF):
            j = i * NBUF + b
            pltpu.make_async_copy(rows_hbm.at[si_v.at[j]],
                                  bufs[b], gsems[b]).wait()
            pltpu.async_copy(bufs[b], acc_sh.at[di_v.at[j]], ssems[b],
                             add=True)
        for b in range(NBUF):
            j = i * NBUF + b

            @pl.when(j + NBUF < cps)
            def _():
                pltpu.make_async_copy(bufs[b], acc_sh.at[di_v.at[j]],
                                      ssems[b]).wait()
                pltpu.async_copy(rows_hbm.at[si_v.at[j + NBUF]],
                                 bufs[b], gsems[b])
        return carry

    lax.fori_loop(0, cps // NBUF, body, 0)
    for b in range(NBUF):
        j = cps - NBUF + b
        pltpu.make_async_copy(bufs[b], acc_sh.at[di_v.at[j]],
                              ssems[b]).wait()


def _make_prop_kernel(width, cps, nphases, feature_split):
    nrows = 2 * nphases if feature_split else nphases
    sems = [pltpu.SemaphoreType.DMA] * (2 * NBUF)

    @functools.partial(
        pl.kernel,
        out_type=[jax.ShapeDtypeStruct((2 * NP, width), jnp.float32)
                  for _ in range(nphases)],
        mesh=_mesh,
        compiler_params=pltpu.CompilerParams(use_tc_tiling_on_sc=False),
        scratch_types=[
            pltpu.VMEM((cps, CHUNK), jnp.int32),
            pltpu.VMEM((cps, CHUNK), jnp.int32),
            pltpu.VMEM((NBUF, CHUNK, width), jnp.float32),
            pltpu.VMEM_SHARED((NP, width), jnp.float32),
        ] + sems,
    )
    def prop(*args):
        rows_list = args[:nrows]
        src_hbm, dst_hbm, zeros_hbm = args[nrows:nrows + 3]
        outs = args[nrows + 3:nrows + 3 + nphases]
        si_v, di_v, buf_v, acc_sh = args[nrows + 3 + nphases:][:4]
        dsems = args[nrows + 7 + nphases:]
        gsems, ssems = dsems[:NBUF], dsems[NBUF:]
        c = lax.axis_index("c")
        s = lax.axis_index("s")
        row0 = s * cps if feature_split else (c * NS + s) * cps
        pltpu.sync_copy(zeros_hbm, acc_sh.at[pl.ds(s * RS, RS)])
        pltpu.sync_copy(src_hbm.at[pl.ds(row0, cps)], si_v)
        pltpu.sync_copy(dst_hbm.at[pl.ds(row0, cps)], di_v)
        plsc.subcore_barrier()

        for p in range(nphases):
            if feature_split:
                @pl.when(c == 0)
                def _():
                    _edge_loop(rows_list[2 * p], si_v, di_v, acc_sh, buf_v,
                               gsems, ssems, cps)

                @pl.when(c == 1)
                def _():
                    _edge_loop(rows_list[2 * p + 1], si_v, di_v, acc_sh,
                               buf_v, gsems, ssems, cps)
            else:
                _edge_loop(rows_list[p], si_v, di_v, acc_sh, buf_v,
                           gsems, ssems, cps)
            plsc.subcore_barrier()
            pltpu.sync_copy(acc_sh.at[pl.ds(s * RS, RS)],
                            outs[p].at[pl.ds(c * NP + s * RS, RS)])
            if p + 1 < nphases:
                pltpu.sync_copy(zeros_hbm, acc_sh.at[pl.ds(s * RS, RS)])
                plsc.subcore_barrier()

    return prop


_l1_kernel = _make_prop_kernel(W1W, CH1, nphases=2, feature_split=True)
_l2_kernel = _make_prop_kernel(W2, CH2, nphases=1, feature_split=False)


def _dinv_of(d0, d1):
    return lax.rsqrt(d0[:, 0:1] + d1[:, 0:1] + 1.0)


def _prep_body(xa_ref, xb_ref, d0_ref, d1_ref, o0, o1, o2, o3):
    dinv = _dinv_of(d0_ref, d1_ref)
    o0[...] = xa_ref[:, 0:W1W] * dinv
    o1[...] = xa_ref[:, W1W:2 * W1W] * dinv
    o2[...] = xb_ref[:, 0:W1W] * dinv
    o3[...] = xb_ref[:, W1W:2 * W1W] * dinv


def _mm_body(q0, q1, q2, q3, x0, x1, x2, x3, d0, d1, w0, w1, o_ref):
    dinv = _dinv_of(d0, d1)
    f = jnp.float32
    h = None
    for k, (q, xq) in enumerate(((q0, x0), (q1, x1), (q2, x2), (q3, x3))):
        p = (q[...] + xq[...]) * dinv
        t = jnp.dot(p, w0[64 * k:64 * (k + 1), :], preferred_element_type=f)
        h = t if h is None else h + t
    h = jnp.maximum(h, 0.0)
    g = jnp.dot(h, w1[...], preferred_element_type=f)
    o_ref[...] = g * dinv


def _fin_body(a0, a1, g, d0, d1, o_ref):
    t = (a0[...] + a1[...] + g[...]) * _dinv_of(d0, d1)
    col = lax.broadcasted_iota(jnp.int32, (RB, W2), 1)
    t = jnp.where(col < NL, t, -jnp.inf)
    m = jnp.max(t, axis=1, keepdims=True)
    e = jnp.exp(t - m)
    o_ref[...] = e / jnp.sum(e, axis=1, keepdims=True)


def kernel(x, edge_index, W0, W1):
    f32 = jnp.float32
    src = edge_index[0].astype(jnp.int32)
    dst = edge_index[1].astype(jnp.int32)

    spread = jnp.arange(PAD, dtype=jnp.int32) % (NP - N)
    pad_rows = N + spread
    src2d = jnp.concatenate([src, pad_rows]).reshape(ER, CHUNK)
    dst2d = jnp.concatenate([dst, pad_rows]).reshape(ER, CHUNK)

    W1p = jnp.pad(W1, ((0, 0), (0, W2 - NL)))

    ones_deg = jnp.ones((CHUNK, 16), f32)
    zeros_deg = jnp.zeros((RS, 16), f32)
    zeros_w1 = jnp.zeros((RS, W1W), f32)
    zeros_w2 = jnp.zeros((RS, W2), f32)

    degs = _deg_kernel(dst2d, ones_deg, zeros_deg)
    d0, d1 = degs[:NP], degs[NP:]

    xq = pl.pallas_call(
        _prep_body,
        grid=(NPB,),
        in_specs=[
            pl.BlockSpec((RB, 128), lambda i: (i, 0)),
            pl.BlockSpec((RB, 128), lambda i: (i, 1)),
            pl.BlockSpec((RB, 16), lambda i: (i, 0)),
            pl.BlockSpec((RB, 16), lambda i: (i, 0)),
        ],
        out_specs=[pl.BlockSpec((RB, W1W), lambda i: (i, 0))] * 4,
        out_shape=[jax.ShapeDtypeStruct((NP, W1W), f32)] * 4,
    )(x, x, d0, d1)

    acc1P0, acc1P1 = _l1_kernel(xq[0], xq[1], xq[2], xq[3],
                                src2d, dst2d, zeros_w1)

    gs = pl.pallas_call(
        _mm_body,
        grid=(NPB,),
        in_specs=[
            pl.BlockSpec((RB, W1W), lambda i: (i, 0)),
            pl.BlockSpec((RB, W1W), lambda i: (NPB + i, 0)),
            pl.BlockSpec((RB, W1W), lambda i: (i, 0)),
            pl.BlockSpec((RB, W1W), lambda i: (NPB + i, 0)),
            pl.BlockSpec((RB, W1W), lambda i: (i, 0)),
            pl.BlockSpec((RB, W1W), lambda i: (i, 0)),
            pl.BlockSpec((RB, W1W), lambda i: (i, 0)),
            pl.BlockSpec((RB, W1W), lambda i: (i, 0)),
            pl.BlockSpec((RB, 16), lambda i: (i, 0)),
            pl.BlockSpec((RB, 16), lambda i: (i, 0)),
            pl.BlockSpec((D_IN, D_HID), lambda i: (0, 0)),
            pl.BlockSpec((D_HID, W2), lambda i: (0, 0)),
        ],
        out_specs=pl.BlockSpec((RB, W2), lambda i: (i, 0)),
        out_shape=jax.ShapeDtypeStruct((NP, W2), f32),
    )(acc1P0, acc1P0, acc1P1, acc1P1, xq[0], xq[1], xq[2], xq[3],
      d0, d1, W0, W1p)

    (acc2,) = _l2_kernel(gs, src2d, dst2d, zeros_w2)

    res = pl.pallas_call(
        _fin_body,
        grid=(NPB,),
        in_specs=[
            pl.BlockSpec((RB, W2), lambda i: (i, 0)),
            pl.BlockSpec((RB, W2), lambda i: (NPB + i, 0)),
            pl.BlockSpec((RB, W2), lambda i: (i, 0)),
            pl.BlockSpec((RB, 16), lambda i: (i, 0)),
            pl.BlockSpec((RB, 16), lambda i: (i, 0)),
        ],
        out_specs=pl.BlockSpec((RB, W2), lambda i: (i, 0)),
        out_shape=jax.ShapeDtypeStruct((NP, W2), f32),
    )(acc2, acc2, gs, d0, d1)

    return res[:N, :NL]


# speedup vs baseline: 22.7782x; 1.0439x over previous
import functools

import jax
import jax.numpy as jnp
from jax import lax
from jax.experimental import pallas as pl
from jax.experimental.pallas import tpu as pltpu
from jax.experimental.pallas import tpu_sc as plsc

N = 10000
D_IN = 256
D_HID = 512
NL = 40
W1W = 64
W2 = 48
E = 160000

NC = 2
NS = 16
NP = 10240
RS = NP // NS
EP = 163840
PAD = EP - E
CHUNK = 128
ER = EP // CHUNK
CH1 = EP // NS // CHUNK
CH2 = EP // (NC * NS) // CHUNK
NBUF = 8

RB = 1024
NPB = NP // RB

_mesh = plsc.VectorSubcoreMesh(core_axis_name="c", subcore_axis_name="s")


@functools.partial(
    pl.kernel,
    out_type=jax.ShapeDtypeStruct((2 * NP, 16), jnp.float32),
    mesh=_mesh,
    compiler_params=pltpu.CompilerParams(use_tc_tiling_on_sc=False),
    scratch_types=[
        pltpu.VMEM((CH2, CHUNK), jnp.int32),
        pltpu.VMEM((CHUNK, 16), jnp.float32),
        pltpu.VMEM_SHARED((NP, 16), jnp.float32),
        pltpu.SemaphoreType.DMA,
        pltpu.SemaphoreType.DMA,
        pltpu.SemaphoreType.DMA,
        pltpu.SemaphoreType.DMA,
    ],
)
def _deg_kernel(dst_hbm, ones_hbm, zeros_hbm, out_hbm, idx_v, ones_v, acc_sh,
                sm0, sm1, sm2, sm3):
    c = lax.axis_index("c")
    s = lax.axis_index("s")
    w = c * NS + s
    pltpu.sync_copy(zeros_hbm, acc_sh.at[pl.ds(s * RS, RS)])
    pltpu.sync_copy(dst_hbm.at[pl.ds(w * CH2, CH2)], idx_v)
    pltpu.sync_copy(ones_hbm, ones_v)
    plsc.subcore_barrier()

    sms = [sm0, sm1, sm2, sm3]
    for j in range(CH2):
        pltpu.async_copy(ones_v, acc_sh.at[idx_v.at[j]], sms[j % 4],
                         add=True)
    for j in range(CH2):
        pltpu.make_async_copy(ones_v, acc_sh.at[idx_v.at[j]],
                              sms[j % 4]).wait()
    plsc.subcore_barrier()
    pltpu.sync_copy(acc_sh.at[pl.ds(s * RS, RS)],
                    out_hbm.at[pl.ds(c * NP + s * RS, RS)])


def _edge_loop(rows_hbm, si_v, di_v, acc_sh, buf_v, gsems, ssems, cps):
    bufs = [buf_v.at[b] for b in range(NBUF)]

    for b in range(NBUF):
        pltpu.async_copy(rows_hbm.at[si_v.at[b]], bufs[b], gsems[b])

    def body(i, carry):
        for b in range(NBUF):
            j = i * NBUF + b
            pltpu.make_async_copy(rows_hbm.at[si_v.at[j]],
                                  bufs[b], gsems[b]).wait()
            pltpu.async_copy(bufs[b], acc_sh.at[di_v.at[j]], ssems[b],
                             add=True)
        for b in range(NBUF):
            j = i * NBUF + b

            @pl.when(j + NBUF < cps)
            def _():
                pltpu.make_async_copy(bufs[b], acc_sh.at[di_v.at[j]],
                                      ssems[b]).wait()
                pltpu.async_copy(rows_hbm.at[si_v.at[j + NBUF]],
                                 bufs[b], gsems[b])
        return carry

    lax.fori_loop(0, cps // NBUF, body, 0)
    for b in range(NBUF):
        j = cps - NBUF + b
        pltpu.make_async_copy(bufs[b], acc_sh.at[di_v.at[j]],
                              ssems[b]).wait()


def _make_prop_kernel(width, cps, nphases, feature_split):
    nrows = 2 * nphases if feature_split else nphases
    sems = [pltpu.SemaphoreType.DMA] * (2 * NBUF)

    @functools.partial(
        pl.kernel,
        out_type=[jax.ShapeDtypeStruct((2 * NP, width), jnp.float32)
                  for _ in range(nphases)],
        mesh=_mesh,
        compiler_params=pltpu.CompilerParams(use_tc_tiling_on_sc=False),
        scratch_types=[
            pltpu.VMEM((cps, CHUNK), jnp.int32),
            pltpu.VMEM((cps, CHUNK), jnp.int32),
            pltpu.VMEM((NBUF, CHUNK, width), jnp.float32),
            pltpu.VMEM_SHARED((NP, width), jnp.float32),
        ] + sems,
    )
    def prop(*args):
        rows_list = args[:nrows]
        src_hbm, dst_hbm, zeros_hbm = args[nrows:nrows + 3]
        outs = args[nrows + 3:nrows + 3 + nphases]
        si_v, di_v, buf_v, acc_sh = args[nrows + 3 + nphases:][:4]
        dsems = args[nrows + 7 + nphases:]
        gsems, ssems = dsems[:NBUF], dsems[NBUF:]
        c = lax.axis_index("c")
        s = lax.axis_index("s")
        row0 = s * cps if feature_split else (c * NS + s) * cps
        pltpu.sync_copy(zeros_hbm, acc_sh.at[pl.ds(s * RS, RS)])
        pltpu.sync_copy(src_hbm.at[pl.ds(row0, cps)], si_v)
        pltpu.sync_copy(dst_hbm.at[pl.ds(row0, cps)], di_v)
        plsc.subcore_barrier()

        for p in range(nphases):
            if feature_split:
                @pl.when(c == 0)
                def _():
                    _edge_loop(rows_list[2 * p], si_v, di_v, acc_sh, buf_v,
                               gsems, ssems, cps)

                @pl.when(c == 1)
                def _():
                    _edge_loop(rows_list[2 * p + 1], si_v, di_v, acc_sh,
                               buf_v, gsems, ssems, cps)
            else:
                _edge_loop(rows_list[p], si_v, di_v, acc_sh, buf_v,
                           gsems, ssems, cps)
            plsc.subcore_barrier()
            pltpu.sync_copy(acc_sh.at[pl.ds(s * RS, RS)],
                            outs[p].at[pl.ds(c * NP + s * RS, RS)])
            if p + 1 < nphases:
                pltpu.sync_copy(zeros_hbm, acc_sh.at[pl.ds(s * RS, RS)])
                plsc.subcore_barrier()

    return prop


_l1_kernel = _make_prop_kernel(W1W, CH1, nphases=2, feature_split=True)
_l2_kernel = _make_prop_kernel(W2, CH2, nphases=1, feature_split=False)


def _dinv_of(d0, d1):
    return lax.rsqrt(d0[:, 0:1] + d1[:, 0:1] + 1.0)


def _prep_body(xa_ref, xb_ref, d0_ref, d1_ref, o0, o1, o2, o3):
    dinv = _dinv_of(d0_ref, d1_ref)
    o0[...] = xa_ref[:, 0:W1W] * dinv
    o1[...] = xa_ref[:, W1W:2 * W1W] * dinv
    o2[...] = xb_ref[:, 0:W1W] * dinv
    o3[...] = xb_ref[:, W1W:2 * W1W] * dinv


def _mm_body(q0, q1, q2, q3, x0, x1, x2, x3, d0, d1, w0, w1, o_ref):
    dinv = _dinv_of(d0, d1)
    f = jnp.float32
    p = jnp.concatenate(
        [(q[...] + xq[...]) * dinv
         for q, xq in ((q0, x0), (q1, x1), (q2, x2), (q3, x3))], axis=1)
    h = jnp.maximum(jnp.dot(p, w0[...], preferred_element_type=f), 0.0)
    g = jnp.dot(h, w1[...], preferred_element_type=f)
    o_ref[...] = g * dinv


def _fin_body(a0, a1, g, d0, d1, o_ref):
    t = (a0[...] + a1[...] + g[...]) * _dinv_of(d0, d1)
    col = lax.broadcasted_iota(jnp.int32, (RB, W2), 1)
    t = jnp.where(col < NL, t, -jnp.inf)
    m = jnp.max(t, axis=1, keepdims=True)
    e = jnp.exp(t - m)
    o_ref[...] = e / jnp.sum(e, axis=1, keepdims=True)


def kernel(x, edge_index, W0, W1):
    f32 = jnp.float32
    src = edge_index[0].astype(jnp.int32)
    dst = edge_index[1].astype(jnp.int32)

    spread = jnp.arange(PAD, dtype=jnp.int32) % (NP - N)
    pad_rows = N + spread
    src2d = jnp.concatenate([src, pad_rows]).reshape(ER, CHUNK)
    dst2d = jnp.concatenate([dst, pad_rows]).reshape(ER, CHUNK)

    W1p = jnp.pad(W1, ((0, 0), (0, W2 - NL)))

    ones_deg = jnp.ones((CHUNK, 16), f32)
    zeros_deg = jnp.zeros((RS, 16), f32)
    zeros_w1 = jnp.zeros((RS, W1W), f32)
    zeros_w2 = jnp.zeros((RS, W2), f32)

    degs = _deg_kernel(dst2d, ones_deg, zeros_deg)

    xq = pl.pallas_call(
        _prep_body,
        grid=(NPB,),
        in_specs=[
            pl.BlockSpec((RB, 128), lambda i: (i, 0)),
            pl.BlockSpec((RB, 128), lambda i: (i, 1)),
            pl.BlockSpec((RB, 16), lambda i: (i, 0)),
            pl.BlockSpec((RB, 16), lambda i: (NPB + i, 0)),
        ],
        out_specs=[pl.BlockSpec((RB, W1W), lambda i: (i, 0))] * 4,
        out_shape=[jax.ShapeDtypeStruct((NP, W1W), f32)] * 4,
    )(x, x, degs, degs)

    acc1P0, acc1P1 = _l1_kernel(xq[0], xq[1], xq[2], xq[3],
                                src2d, dst2d, zeros_w1)

    gs = pl.pallas_call(
        _mm_body,
        grid=(NPB,),
        in_specs=[
            pl.BlockSpec((RB, W1W), lambda i: (i, 0)),
            pl.BlockSpec((RB, W1W), lambda i: (NPB + i, 0)),
            pl.BlockSpec((RB, W1W), lambda i: (i, 0)),
            pl.BlockSpec((RB, W1W), lambda i: (NPB + i, 0)),
            pl.BlockSpec((RB, W1W), lambda i: (i, 0)),
            pl.BlockSpec((RB, W1W), lambda i: (i, 0)),
            pl.BlockSpec((RB, W1W), lambda i: (i, 0)),
            pl.BlockSpec((RB, W1W), lambda i: (i, 0)),
            pl.BlockSpec((RB, 16), lambda i: (i, 0)),
            pl.BlockSpec((RB, 16), lambda i: (NPB + i, 0)),
            pl.BlockSpec((D_IN, D_HID), lambda i: (0, 0)),
            pl.BlockSpec((D_HID, W2), lambda i: (0, 0)),
        ],
        out_specs=pl.BlockSpec((RB, W2), lambda i: (i, 0)),
        out_shape=jax.ShapeDtypeStruct((NP, W2), f32),
    )(acc1P0, acc1P0, acc1P1, acc1P1, xq[0], xq[1], xq[2], xq[3],
      degs, degs, W0, W1p)

    (acc2,) = _l2_kernel(gs, src2d, dst2d, zeros_w2)

    res = pl.pallas_call(
        _fin_body,
        grid=(NPB,),
        in_specs=[
            pl.BlockSpec((RB, W2), lambda i: (i, 0)),
            pl.BlockSpec((RB, W2), lambda i: (NPB + i, 0)),
            pl.BlockSpec((RB, W2), lambda i: (i, 0)),
            pl.BlockSpec((RB, 16), lambda i: (i, 0)),
            pl.BlockSpec((RB, 16), lambda i: (NPB + i, 0)),
        ],
        out_specs=pl.BlockSpec((RB, W2), lambda i: (i, 0)),
        out_shape=jax.ShapeDtypeStruct((NP, W2), f32),
    )(acc2, acc2, gs, degs, degs)

    return res[:N, :NL]
